# Initial kernel scaffold; baseline (speedup 1.0000x reference)
import jax
import jax.numpy as jnp
from jax.experimental import pallas as pl


def kernel(x_user, x_movie, edge_label_index):
    raise NotImplementedError("write your pallas kernel here")


# speedup vs baseline: 1.0169x; 1.0169x over previous
import functools

import jax
import jax.numpy as jnp
from jax import lax
from jax.experimental import pallas as pl
from jax.experimental.pallas import tpu as pltpu
from jax.experimental.pallas import tpu_sc as plsc

E = 320000
D = 128
NC, NS = 2, 16
NW = NC * NS
PER_W = E // NW
CHUNK = 80
NCHUNK = PER_W // CHUNK
GROUPS = CHUNK // 16
UNROLL = 8


def _body(xu_hbm, xm_hbm, uidx_hbm, midx_hbm, out_hbm,
          uidx_v, midx_v, out_v, ru_v, rm_v, sem_u, sem_m):
    wid = lax.axis_index("s") * NC + lax.axis_index("c")
    base = wid * PER_W
    pltpu.sync_copy(uidx_hbm.at[pl.ds(base, PER_W)], uidx_v)
    pltpu.sync_copy(midx_hbm.at[pl.ds(base, PER_W)], midx_v)

    lane = lax.iota(jnp.int32, 16)

    def chunk_body(ci, _):
        off = ci * CHUNK
        cu = pltpu.async_copy(xu_hbm.at[uidx_v.at[pl.ds(off, CHUNK)]], ru_v, sem_u)
        cm = pltpu.async_copy(xm_hbm.at[midx_v.at[pl.ds(off, CHUNK)]], rm_v, sem_m)
        cu.wait()
        cm.wait()

        def group_body(g, _):
            e_vec = g * 16 + lane

            def d_body(di, carry):
                acc, db = carry
                for j in range(UNROLL):
                    dv = db + j
                    gu = plsc.load_gather(ru_v, [e_vec, dv])
                    gm = plsc.load_gather(rm_v, [e_vec, dv])
                    acc = acc + gu * gm
                return acc, db + UNROLL

            acc, _ = lax.fori_loop(
                0, D // UNROLL, d_body,
                (jnp.zeros((16,), jnp.float32), jnp.zeros((16,), jnp.int32)))
            out_v[pl.ds(off + g * 16, 16)] = acc
            return 0

        lax.fori_loop(0, GROUPS, group_body, 0)
        return 0

    lax.fori_loop(0, NCHUNK, chunk_body, 0)
    pltpu.sync_copy(out_v, out_hbm.at[pl.ds(base, PER_W)])


@jax.jit
def _scores(x_user, x_movie, u_idx, m_idx):
    mesh = plsc.VectorSubcoreMesh(core_axis_name="c", subcore_axis_name="s")
    return pl.kernel(
        _body,
        out_type=jax.ShapeDtypeStruct((E,), jnp.float32),
        mesh=mesh,
        compiler_params=pltpu.CompilerParams(needs_layout_passes=False),
        scratch_types=[
            pltpu.VMEM((PER_W,), jnp.int32),
            pltpu.VMEM((PER_W,), jnp.int32),
            pltpu.VMEM((PER_W,), jnp.float32),
            pltpu.VMEM((CHUNK, D), jnp.float32),
            pltpu.VMEM((CHUNK, D), jnp.float32),
            pltpu.SemaphoreType.DMA,
            pltpu.SemaphoreType.DMA,
        ],
    )(x_user, x_movie, u_idx, m_idx)


def kernel(x_user, x_movie, edge_label_index):
    u_idx = edge_label_index[0]
    m_idx = edge_label_index[1]
    return _scores(x_user, x_movie, u_idx, m_idx)


# speedup vs baseline: 1.1515x; 1.1324x over previous
import jax
import jax.numpy as jnp
from jax import lax
from jax.experimental import pallas as pl
from jax.experimental.pallas import tpu as pltpu
from jax.experimental.pallas import tpu_sc as plsc

E = 320000
D = 128
NC, NS = 2, 16
NW = NC * NS
PER_W = E // NW
CHUNK = 80
NCHUNK = PER_W // CHUNK
GROUPS = CHUNK // 16
UNROLL = 8
NBUF = 4


def _body(xu_hbm, xm_hbm, uidx_hbm, midx_hbm, out_hbm,
          uidx_v, midx_v, out_v, ru_v, rm_v, *sems):
    sem_u = sems[:NBUF]
    sem_m = sems[NBUF:]
    wid = lax.axis_index("s") * NC + lax.axis_index("c")
    base = wid * PER_W
    pltpu.sync_copy(uidx_hbm.at[pl.ds(base, PER_W)], uidx_v)
    pltpu.sync_copy(midx_hbm.at[pl.ds(base, PER_W)], midx_v)

    lane = lax.iota(jnp.int32, 16)

    def issue(c, b):
        off = c * CHUNK
        pltpu.async_copy(xu_hbm.at[uidx_v.at[pl.ds(off, CHUNK)]],
                         ru_v.at[pl.ds(b * CHUNK, CHUNK)], sem_u[b])
        pltpu.async_copy(xm_hbm.at[midx_v.at[pl.ds(off, CHUNK)]],
                         rm_v.at[pl.ds(b * CHUNK, CHUNK)], sem_m[b])

    def wait_slot(b):
        pltpu.make_async_copy(xu_hbm.at[uidx_v.at[pl.ds(0, CHUNK)]],
                              ru_v.at[pl.ds(b * CHUNK, CHUNK)], sem_u[b]).wait()
        pltpu.make_async_copy(xm_hbm.at[midx_v.at[pl.ds(0, CHUNK)]],
                              rm_v.at[pl.ds(b * CHUNK, CHUNK)], sem_m[b]).wait()

    def compute(c, b):
        def group_body(g, _):
            e_vec = b * CHUNK + g * 16 + lane

            def d_body(di, carry):
                acc, db = carry
                for j in range(UNROLL):
                    dv = db + j
                    gu = plsc.load_gather(ru_v, [e_vec, dv])
                    gm = plsc.load_gather(rm_v, [e_vec, dv])
                    acc = acc + gu * gm
                return acc, db + UNROLL

            acc, _ = lax.fori_loop(
                0, D // UNROLL, d_body,
                (jnp.zeros((16,), jnp.float32), jnp.zeros((16,), jnp.int32)))
            out_v[pl.ds(c * CHUNK + g * 16, 16)] = acc
            return 0

        lax.fori_loop(0, GROUPS, group_body, 0)

    for b in range(NBUF):
        issue(b, b)

    def t_body(t, _):
        for b in range(NBUF):
            c = t * NBUF + b
            wait_slot(b)
            compute(c, b)

            @pl.when(c + NBUF < NCHUNK)
            def _():
                issue(c + NBUF, b)
        return 0

    lax.fori_loop(0, NCHUNK // NBUF, t_body, 0)
    for c in range(NCHUNK - NCHUNK % NBUF, NCHUNK):
        wait_slot(c % NBUF)
        compute(c, c % NBUF)

    pltpu.sync_copy(out_v, out_hbm.at[pl.ds(base, PER_W)])


@jax.jit
def _scores(x_user, x_movie, u_idx, m_idx):
    mesh = plsc.VectorSubcoreMesh(core_axis_name="c", subcore_axis_name="s")
    return pl.kernel(
        _body,
        out_type=jax.ShapeDtypeStruct((E,), jnp.float32),
        mesh=mesh,
        compiler_params=pltpu.CompilerParams(needs_layout_passes=False),
        scratch_types=[
            pltpu.VMEM((PER_W,), jnp.int32),
            pltpu.VMEM((PER_W,), jnp.int32),
            pltpu.VMEM((PER_W,), jnp.float32),
            pltpu.VMEM((NBUF * CHUNK, D), jnp.float32),
            pltpu.VMEM((NBUF * CHUNK, D), jnp.float32),
        ] + [pltpu.SemaphoreType.DMA] * (2 * NBUF),
    )(x_user, x_movie, u_idx, m_idx)


def kernel(x_user, x_movie, edge_label_index):
    u_idx = edge_label_index[0]
    m_idx = edge_label_index[1]
    return _scores(x_user, x_movie, u_idx, m_idx)
